# baseline (speedup 1.0000x reference)
import functools

import jax
import jax.numpy as jnp
from jax.experimental import pallas as pl
from jax.experimental.pallas import tpu as pltpu

_CONF_THRES = 0.25
_IOU_THRES = 0.45
_RATIO = 0.02
_MAX_DET = 300
_MAX_WH = 7680.0
_IMG_SIZE = 640.0

_N = 20000
_ROWS = 160
_LANES = 128
_NPAD = _ROWS * _LANES


def _nms_body(inp_ref, out_ref):
    f32 = jnp.float32
    i32 = jnp.int32
    conf = inp_ref[4]
    cls = jnp.zeros((_ROWS, _LANES), f32)
    for c in range(1, 80):
        s = inp_ref[4 + c]
        upd = s > conf
        cls = jnp.where(upd, f32(c), cls)
        conf = jnp.where(upd, s, conf)

    xc = inp_ref[0]
    yc = inp_ref[1]
    hw = inp_ref[2] * 0.5
    hh = inp_ref[3] * 0.5
    x1 = xc - hw
    y1 = yc - hh
    x2 = xc + hw
    y2 = yc + hh
    off = cls * _MAX_WH
    ox1 = x1 + off
    oy1 = y1 + off
    ox2 = x2 + off
    oy2 = y2 + off
    tarea = ((x2 - x1) * (y2 - y1)) * _IOU_THRES

    valid = conf > _CONF_THRES
    scores0 = jnp.where(valid, conf, f32(-1.0))

    stack = jnp.concatenate([x1, y1, x2, y2, cls], axis=1)

    flat = (jax.lax.broadcasted_iota(i32, (_ROWS, _LANES), 0) * _LANES
            + jax.lax.broadcasted_iota(i32, (_ROWS, _LANES), 1))
    lane8 = jax.lax.broadcasted_iota(i32, (8, _LANES), 1)
    row8 = jax.lax.broadcasted_iota(i32, (8, _LANES), 0)

    def tmax(x):
        x = jnp.maximum(x[0:80], x[80:160])
        x = jnp.maximum(x[0:40], x[40:80])
        return jnp.max(x, keepdims=True)

    def tmin(x):
        x = jnp.minimum(x[0:80], x[80:160])
        x = jnp.minimum(x[0:40], x[40:80])
        return jnp.min(x, keepdims=True)

    def body(i, state):
        scores, num, pconf, pcontrib = state
        best = tmax(scores)
        is_valid = best > 0.0
        m1 = scores >= best
        fidx = tmin(jnp.where(m1, flat, jnp.int32(2**30)))
        mask = flat == fidx
        m5 = jnp.concatenate([mask] * 5, axis=1)
        g = jnp.where(m5, stack, 0.0)
        g = g[0:80] + g[80:160]
        g = g[0:40] + g[40:80]
        gs = jnp.sum(g, axis=0, keepdims=True)
        bx1 = jnp.sum(gs[:, 0:128], keepdims=True)
        by1 = jnp.sum(gs[:, 128:256], keepdims=True)
        bx2 = jnp.sum(gs[:, 256:384], keepdims=True)
        by2 = jnp.sum(gs[:, 384:512], keepdims=True)
        bcl = jnp.sum(gs[:, 512:640], keepdims=True)
        boff = bcl * _MAX_WH
        bax = bx2 - bx1
        bay = by2 - by1
        rhs0 = _IOU_THRES * (bax * bay + 1e-7)

        ix1 = jnp.maximum(bx1 + boff, ox1)
        iy1 = jnp.maximum(by1 + boff, oy1)
        ix2 = jnp.minimum(bx2 + boff, ox2)
        iy2 = jnp.minimum(by2 + boff, oy2)
        inter = jnp.maximum(ix2 - ix1, 0.0) * jnp.maximum(iy2 - iy1, 0.0)
        sup = inter * (1.0 + _IOU_THRES) > rhs0 + tarea
        scores = jnp.where(sup | mask, f32(-1.0), scores)

        num = num + jnp.where(is_valid, f32(1.0), f32(0.0))
        rec = is_valid & (i < 6)
        slotmask = (row8 == 0) & (lane8 == i) & rec
        pconf = jnp.where(slotmask, best, pconf)
        bcontrib = best * ((bx1 + by1 + bx2 + by2) * (1.0 / _IMG_SIZE) + bcl)
        pcontrib = jnp.where(slotmask, bcontrib, pcontrib)
        return scores, num, pconf, pcontrib

    init = (scores0, jnp.zeros((1, 1), f32), jnp.zeros((8, _LANES), f32),
            jnp.zeros((8, _LANES), f32))
    _, num, pconf, pcontrib = jax.lax.fori_loop(0, _MAX_DET, body, init)

    k = jnp.maximum(jnp.int32(1),
                    jnp.floor(num * f32(_RATIO)).astype(i32))
    usemask = (row8 == 0) & (lane8 < k)
    wsum = jnp.sum(jnp.where(usemask, pconf, 0.0), keepdims=True)
    wvsum = jnp.sum(jnp.where(usemask, pcontrib, 0.0), keepdims=True)
    target = wvsum / (2.0 * wsum)
    outv = jnp.where(num > 0.0, target, f32(0.0))
    out_ref[...] = jnp.zeros((8, _LANES), f32) + outv


@jax.jit
def kernel(model_output):
    x = model_output[0]
    xt = jnp.transpose(x)
    xt = jnp.pad(xt, ((0, 0), (0, _NPAD - _N)))
    xt = xt.reshape(84, _ROWS, _LANES)
    out = pl.pallas_call(
        _nms_body,
        out_shape=jax.ShapeDtypeStruct((8, _LANES), jnp.float32),
    )(xt)
    return out[0, 0]


# speedup vs baseline: 1.0890x; 1.0890x over previous
import functools

import jax
import jax.numpy as jnp
from jax.experimental import pallas as pl
from jax.experimental.pallas import tpu as pltpu

_CONF_THRES = 0.25
_IOU_THRES = 0.45
_RATIO = 0.02
_MAX_DET = 300
_MAX_WH = 7680.0
_IMG_SIZE = 640.0

_N = 20000
_ROWS = 160
_LANES = 128
_NPAD = _ROWS * _LANES


def _nms_body(inp_ref, out_ref):
    f32 = jnp.float32
    i32 = jnp.int32
    x = inp_ref[...]
    xt = jnp.transpose(x)
    xt = jnp.concatenate(
        [xt, jnp.zeros((84, _NPAD - _N), f32)], axis=1)
    xt3 = xt.reshape(84, _ROWS, _LANES)
    conf = xt3[4]
    cls = jnp.zeros((_ROWS, _LANES), f32)
    for c in range(1, 80):
        s = xt3[4 + c]
        upd = s > conf
        cls = jnp.where(upd, f32(c), cls)
        conf = jnp.where(upd, s, conf)

    xc = xt3[0]
    yc = xt3[1]
    hw = xt3[2] * 0.5
    hh = xt3[3] * 0.5
    x1 = xc - hw
    y1 = yc - hh
    x2 = xc + hw
    y2 = yc + hh
    off = cls * _MAX_WH
    ox1 = x1 + off
    oy1 = y1 + off
    ox2 = x2 + off
    oy2 = y2 + off
    tarea = ((x2 - x1) * (y2 - y1)) * _IOU_THRES

    valid = conf > _CONF_THRES
    scores0 = jnp.where(valid, conf, f32(-1.0))

    stack = jnp.concatenate([x1, y1, x2, y2, cls], axis=1)

    flat = (jax.lax.broadcasted_iota(i32, (_ROWS, _LANES), 0) * _LANES
            + jax.lax.broadcasted_iota(i32, (_ROWS, _LANES), 1))
    lane8 = jax.lax.broadcasted_iota(i32, (8, _LANES), 1)
    row8 = jax.lax.broadcasted_iota(i32, (8, _LANES), 0)

    def tmax(x):
        x = jnp.maximum(x[0:80], x[80:160])
        x = jnp.maximum(x[0:40], x[40:80])
        return jnp.max(x, keepdims=True)

    def tmin(x):
        x = jnp.minimum(x[0:80], x[80:160])
        x = jnp.minimum(x[0:40], x[40:80])
        return jnp.min(x, keepdims=True)

    def body(i, state):
        scores, num, pconf, pcontrib = state
        best = tmax(scores)
        is_valid = best > 0.0
        m1 = scores >= best
        fidx = tmin(jnp.where(m1, flat, jnp.int32(2**30)))
        mask = flat == fidx
        m5 = jnp.concatenate([mask] * 5, axis=1)
        g = jnp.where(m5, stack, 0.0)
        g = g[0:80] + g[80:160]
        g = g[0:40] + g[40:80]
        gs = jnp.sum(g, axis=0, keepdims=True)
        bx1 = jnp.sum(gs[:, 0:128], keepdims=True)
        by1 = jnp.sum(gs[:, 128:256], keepdims=True)
        bx2 = jnp.sum(gs[:, 256:384], keepdims=True)
        by2 = jnp.sum(gs[:, 384:512], keepdims=True)
        bcl = jnp.sum(gs[:, 512:640], keepdims=True)
        boff = bcl * _MAX_WH
        bax = bx2 - bx1
        bay = by2 - by1
        rhs0 = _IOU_THRES * (bax * bay + 1e-7)

        ix1 = jnp.maximum(bx1 + boff, ox1)
        iy1 = jnp.maximum(by1 + boff, oy1)
        ix2 = jnp.minimum(bx2 + boff, ox2)
        iy2 = jnp.minimum(by2 + boff, oy2)
        inter = jnp.maximum(ix2 - ix1, 0.0) * jnp.maximum(iy2 - iy1, 0.0)
        sup = inter * (1.0 + _IOU_THRES) > rhs0 + tarea
        scores = jnp.where(sup | mask, f32(-1.0), scores)

        num = num + jnp.where(is_valid, f32(1.0), f32(0.0))
        rec = is_valid & (i < 6)
        slotmask = (row8 == 0) & (lane8 == i) & rec
        pconf = jnp.where(slotmask, best, pconf)
        bcontrib = best * ((bx1 + by1 + bx2 + by2) * (1.0 / _IMG_SIZE) + bcl)
        pcontrib = jnp.where(slotmask, bcontrib, pcontrib)
        return scores, num, pconf, pcontrib

    init = (scores0, jnp.zeros((1, 1), f32), jnp.zeros((8, _LANES), f32),
            jnp.zeros((8, _LANES), f32))
    _, num, pconf, pcontrib = jax.lax.fori_loop(0, _MAX_DET, body, init)

    k = jnp.maximum(jnp.int32(1),
                    jnp.floor(num * f32(_RATIO)).astype(i32))
    usemask = (row8 == 0) & (lane8 < k)
    wsum = jnp.sum(jnp.where(usemask, pconf, 0.0), keepdims=True)
    wvsum = jnp.sum(jnp.where(usemask, pcontrib, 0.0), keepdims=True)
    target = wvsum / (2.0 * wsum)
    outv = jnp.where(num > 0.0, target, f32(0.0))
    out_ref[...] = jnp.zeros((8, _LANES), f32) + outv


@jax.jit
def kernel(model_output):
    x = model_output[0]
    out = pl.pallas_call(
        _nms_body,
        out_shape=jax.ShapeDtypeStruct((8, _LANES), jnp.float32),
    )(x)
    return out[0, 0]
